# baseline (speedup 1.0000x reference)
import jax
import jax.numpy as jnp
from jax import lax
from jax.experimental import pallas as pl
from jax.experimental.pallas import tpu as pltpu
from jax.experimental.pallas import tpu_sc as plsc

BATCH = 16384
DIM = 16

_NC = 2
_NS = 16
_NW = _NC * _NS
_ROWS_PER_W = BATCH // _NW
_CHUNK = 128
_NCHUNK = _ROWS_PER_W // _CHUNK
_NGROUP = _ROWS_PER_W // 16


def _tt_body(x_hbm, y_hbm, art_hbm, cust_hbm, out_hbm,
             xidx, yidx, xrows, yrows, out_v, sem):
    wid = lax.axis_index("s") * _NC + lax.axis_index("c")
    base = wid * _ROWS_PER_W

    idx_copies = []
    for j in range(_NCHUNK):
        src = pl.ds(base + j * _CHUNK, _CHUNK)
        idx_copies.append(pltpu.make_async_copy(x_hbm.at[src], xidx.at[j], sem))
        idx_copies.append(pltpu.make_async_copy(y_hbm.at[src], yidx.at[j], sem))
    for c in idx_copies:
        c.start()
    for c in idx_copies:
        c.wait()

    row_copies = []
    for j in range(_NCHUNK):
        dst = pl.ds(j * _CHUNK, _CHUNK)
        row_copies.append(
            pltpu.make_async_copy(cust_hbm.at[xidx.at[j]], xrows.at[dst], sem))
        row_copies.append(
            pltpu.make_async_copy(art_hbm.at[yidx.at[j]], yrows.at[dst], sem))
    for c in row_copies:
        c.start()
    for c in row_copies:
        c.wait()

    iota = lax.iota(jnp.int32, 16)

    def group(g, carry):
        rows = g * 16 + iota
        acc = jnp.zeros((16,), jnp.float32)
        for p in range(DIM // 2):
            cols = lax.bitwise_and(iota + p, 7)
            xw = plsc.load_gather(xrows, [rows, cols])
            yw = plsc.load_gather(yrows, [rows, cols])
            xa, xb = plsc.unpack(plsc.bitcast(xw, jnp.bfloat16),
                                 format=plsc.PackFormat.INTERLEAVED)
            ya, yb = plsc.unpack(plsc.bitcast(yw, jnp.bfloat16),
                                 format=plsc.PackFormat.INTERLEAVED)
            acc = acc + xa * ya + xb * yb
        out_v[pl.ds(g * 16, 16)] = acc
        return carry

    lax.fori_loop(0, _NGROUP, group, None)

    pltpu.sync_copy(out_v, out_hbm.at[pl.ds(base, _ROWS_PER_W)])


def kernel(x, y, article_table, customer_table):
    x = x.astype(jnp.int32)
    y = y.astype(jnp.int32)
    art_b = lax.bitcast_convert_type(
        article_table.astype(jnp.bfloat16).reshape(-1, DIM // 2, 2), jnp.int32)
    cust_b = lax.bitcast_convert_type(
        customer_table.astype(jnp.bfloat16).reshape(-1, DIM // 2, 2), jnp.int32)
    mesh = plsc.VectorSubcoreMesh(
        core_axis_name="c", subcore_axis_name="s",
        num_cores=_NC, num_subcores=_NS)
    run = pl.kernel(
        _tt_body,
        out_type=jax.ShapeDtypeStruct((BATCH,), jnp.float32),
        mesh=mesh,
        scratch_types=[
            pltpu.VMEM((_NCHUNK, _CHUNK), jnp.int32),
            pltpu.VMEM((_NCHUNK, _CHUNK), jnp.int32),
            pltpu.VMEM((_ROWS_PER_W, DIM // 2), jnp.int32),
            pltpu.VMEM((_ROWS_PER_W, DIM // 2), jnp.int32),
            pltpu.VMEM((_ROWS_PER_W,), jnp.float32),
            pltpu.SemaphoreType.DMA,
        ],
        compiler_params=pltpu.CompilerParams(
            needs_layout_passes=False, use_tc_tiling_on_sc=False),
    )
    return run(x, y, art_b, cust_b)


# speedup vs baseline: 2.2769x; 2.2769x over previous
import jax
import jax.numpy as jnp
from jax import lax
from jax.experimental import pallas as pl
from jax.experimental.pallas import tpu as pltpu
from jax.experimental.pallas import tpu_sc as plsc

BATCH = 16384
DIM = 16

_NC = 2
_NS = 16
_NW = _NC * _NS
_ROWS_PER_W = BATCH // _NW
_CHUNK = 128
_NCHUNK = _ROWS_PER_W // _CHUNK
_NGROUP = _ROWS_PER_W // 16


def _tt_body(x_hbm, y_hbm, art_hbm, cust_hbm, out_hbm,
             xidx, yidx, xrows, yrows, out_v, sem):
    wid = lax.axis_index("s") * _NC + lax.axis_index("c")
    base = wid * _ROWS_PER_W

    idx_copies = []
    for j in range(_NCHUNK):
        src = pl.ds(base + j * _CHUNK, _CHUNK)
        idx_copies.append(pltpu.make_async_copy(x_hbm.at[src], xidx.at[j], sem))
        idx_copies.append(pltpu.make_async_copy(y_hbm.at[src], yidx.at[j], sem))
    for c in idx_copies:
        c.start()
    for c in idx_copies:
        c.wait()

    row_copies = []
    for j in range(_NCHUNK):
        dst = pl.ds(j * _CHUNK, _CHUNK)
        row_copies.append(
            pltpu.make_async_copy(cust_hbm.at[xidx.at[j]], xrows.at[dst], sem))
        row_copies.append(
            pltpu.make_async_copy(art_hbm.at[yidx.at[j]], yrows.at[dst], sem))
    for c in row_copies:
        c.start()
    for c in row_copies:
        c.wait()

    iota = lax.iota(jnp.int32, 16)

    def group(g, carry):
        rows = g * 16 + iota
        acc = jnp.zeros((16,), jnp.float32)
        for d in range(DIM):
            cols = lax.bitwise_and(iota + d, 15)
            xa = plsc.load_gather(xrows, [rows, cols])
            ya = plsc.load_gather(yrows, [rows, cols])
            acc = acc + xa * ya
        out_v[pl.ds(g * 16, 16)] = acc
        return carry

    lax.fori_loop(0, _NGROUP, group, None)

    pltpu.sync_copy(out_v, out_hbm.at[pl.ds(base, _ROWS_PER_W)])


def kernel(x, y, article_table, customer_table):
    x = x.astype(jnp.int32)
    y = y.astype(jnp.int32)
    mesh = plsc.VectorSubcoreMesh(
        core_axis_name="c", subcore_axis_name="s",
        num_cores=_NC, num_subcores=_NS)
    run = pl.kernel(
        _tt_body,
        out_type=jax.ShapeDtypeStruct((BATCH,), jnp.float32),
        mesh=mesh,
        scratch_types=[
            pltpu.VMEM((_NCHUNK, _CHUNK), jnp.int32),
            pltpu.VMEM((_NCHUNK, _CHUNK), jnp.int32),
            pltpu.VMEM((_ROWS_PER_W, DIM), jnp.float32),
            pltpu.VMEM((_ROWS_PER_W, DIM), jnp.float32),
            pltpu.VMEM((_ROWS_PER_W,), jnp.float32),
            pltpu.SemaphoreType.DMA,
        ],
        compiler_params=pltpu.CompilerParams(
            needs_layout_passes=False, use_tc_tiling_on_sc=False),
    )
    return run(x, y, article_table, customer_table)
